# baseline (speedup 1.0000x reference)
import jax
import jax.numpy as jnp
from jax.experimental import pallas as pl
from jax.experimental.pallas import tpu as pltpu

_N = 8192
_BATCH = 4096
_DIM = 512
_BB = 256
_NB = 2048


def _gate_body(x_ref, w_ref, o_ref, c_ref):
    j = pl.program_id(1)

    @pl.when(j == 0)
    def _compute_count():
        z = jnp.dot(x_ref[...], w_ref[...],
                    preferred_element_type=jnp.float32)
        c_ref[...] = jax.nn.sigmoid(z) * _N

    base = (j * _NB).astype(jnp.float32)
    idx = jax.lax.broadcasted_iota(
        jnp.int32, (_BB, _NB), 1).astype(jnp.float32) + base
    o_ref[...] = jnp.clip(c_ref[...] - idx, 0.0, 1.0)


def kernel(x, w_count):
    grid = (_BATCH // _BB, _N // _NB)
    return pl.pallas_call(
        _gate_body,
        grid=grid,
        in_specs=[
            pl.BlockSpec((_BB, _DIM), lambda i, j: (i, 0)),
            pl.BlockSpec((_DIM, 1), lambda i, j: (0, 0)),
        ],
        out_specs=pl.BlockSpec((_BB, _NB), lambda i, j: (i, j)),
        out_shape=jax.ShapeDtypeStruct((_BATCH, _N), jnp.float32),
        scratch_shapes=[pltpu.VMEM((_BB, 1), jnp.float32)],
    )(x, w_count)


# speedup vs baseline: 1.4299x; 1.4299x over previous
import jax
import jax.numpy as jnp
from jax.experimental import pallas as pl
from jax.experimental.pallas import tpu as pltpu

_N = 8192
_BATCH = 4096
_DIM = 512
_BB = 128


def _gate_body(x_ref, w_ref, o_ref):
    z = jnp.dot(x_ref[...], w_ref[...], preferred_element_type=jnp.float32)
    c = jax.nn.sigmoid(z) * _N
    idx = jax.lax.broadcasted_iota(jnp.int32, (_BB, _N), 1).astype(jnp.float32)
    o_ref[...] = jnp.clip(c - idx, 0.0, 1.0)


def kernel(x, w_count):
    return pl.pallas_call(
        _gate_body,
        grid=(_BATCH // _BB,),
        in_specs=[
            pl.BlockSpec((_BB, _DIM), lambda i: (i, 0)),
            pl.BlockSpec((_DIM, 1), lambda i: (0, 0)),
        ],
        out_specs=pl.BlockSpec((_BB, _N), lambda i: (i, 0)),
        out_shape=jax.ShapeDtypeStruct((_BATCH, _N), jnp.float32),
    )(x, w_count)


# speedup vs baseline: 1.6307x; 1.1404x over previous
import jax
import jax.numpy as jnp
from jax.experimental import pallas as pl
from jax.experimental.pallas import tpu as pltpu

_N = 8192
_BATCH = 4096
_DIM = 512
_BB = 256


def _gate_body(x_ref, w_ref, o_ref):
    z = jnp.dot(x_ref[...], w_ref[...], preferred_element_type=jnp.float32)
    c = jax.nn.sigmoid(z) * _N
    idx = jax.lax.broadcasted_iota(jnp.int32, (_BB, _N), 1).astype(jnp.float32)
    o_ref[...] = jnp.clip(c - idx, 0.0, 1.0)


def kernel(x, w_count):
    return pl.pallas_call(
        _gate_body,
        grid=(_BATCH // _BB,),
        in_specs=[
            pl.BlockSpec((_BB, _DIM), lambda i: (i, 0)),
            pl.BlockSpec((_DIM, 1), lambda i: (0, 0)),
        ],
        out_specs=pl.BlockSpec((_BB, _N), lambda i: (i, 0)),
        out_shape=jax.ShapeDtypeStruct((_BATCH, _N), jnp.float32),
    )(x, w_count)
